# baseline (speedup 1.0000x reference)
import jax
import jax.numpy as jnp
from jax import lax
from jax.experimental import pallas as pl
from jax.experimental.pallas import tpu as pltpu, tpu_sc as plsc

B = 16384
D = 64
NNEG = 20
NC = 2
NS = 16
NW = NC * NS
BPW = B // NW
CH = 32
NCHUNK = BPW // CH
L = 16
GPC = CH // L


def _body(in_table, out_table, in_idx, ctx_idx, neg_idx, pos_out, neg_out,
          in_idx_v, ctx_idx_v, neg_raw_v, neg_idx_t, in_rows, pos_rows,
          neg_rows, pos_v, neg_v, sems):
    wid = lax.axis_index("s") * NC + lax.axis_index("c")

    pltpu.sync_copy(in_idx.at[wid], in_idx_v)
    pltpu.sync_copy(ctx_idx.at[wid], ctx_idx_v)
    pltpu.sync_copy(neg_idx.at[wid], neg_raw_v)

    iota = lax.iota(jnp.int32, L)

    def stage(c, p):
        cvec = jnp.zeros((L,), jnp.int32) + c
        for j in range(NNEG):
            jvec = jnp.full((L,), j, jnp.int32)
            for g in range(GPC):
                rid = iota + (g * L)
                col = plsc.load_gather(neg_raw_v, [cvec, rid, jvec])
                neg_idx_t[p, j, pl.ds(g * L, L)] = col
        cps = [
            pltpu.async_copy(in_table.at[in_idx_v.at[c]], in_rows.at[p],
                             sems.at[p]),
            pltpu.async_copy(out_table.at[ctx_idx_v.at[c]], pos_rows.at[p],
                             sems.at[p]),
        ]
        for j in range(NNEG):
            cps.append(pltpu.async_copy(out_table.at[neg_idx_t.at[p, j]],
                                        neg_rows.at[p, j], sems.at[p]))
        return cps

    def drain(p):
        pltpu.make_async_copy(in_table.at[in_idx_v.at[0]], in_rows.at[p],
                              sems.at[p]).wait()
        pltpu.make_async_copy(out_table.at[ctx_idx_v.at[0]], pos_rows.at[p],
                              sems.at[p]).wait()
        for j in range(NNEG):
            pltpu.make_async_copy(out_table.at[neg_idx_t.at[p, j]],
                                  neg_rows.at[p, j], sems.at[p]).wait()

    stage(0, 0)

    def chunk_body(c, carry):
        p = c & 1
        drain(p)

        @pl.when(c + 1 < NCHUNK)
        def _():
            stage(c + 1, 1 - p)

        pvec = jnp.zeros((L,), jnp.int32) + p

        for g in range(GPC):
            rid = iota + (g * L)

            def d_body(d2, accs):
                for u in range(2):
                    dvec = (iota + (d2 * 2 + u)) & (D - 1)
                    inv = plsc.load_gather(in_rows, [pvec, rid, dvec])
                    pv = plsc.load_gather(pos_rows, [pvec, rid, dvec])
                    new = [accs[0] + inv * pv]
                    for j in range(NNEG):
                        jvec = jnp.full((L,), j, jnp.int32)
                        nv = plsc.load_gather(neg_rows,
                                              [pvec, jvec, rid, dvec])
                        new.append(accs[j + 1] + inv * nv)
                    accs = tuple(new)
                return accs

            zeros = tuple(jnp.zeros((L,), jnp.float32)
                          for _ in range(NNEG + 1))
            accs = lax.fori_loop(0, D // 2, d_body, zeros)

            off = c * CH + g * L
            pos_v[pl.ds(off, L)] = accs[0]
            rid_w = iota + off
            for j in range(NNEG):
                jvec = jnp.full((L,), j, jnp.int32)
                plsc.store_scatter(neg_v, [rid_w, jvec], accs[j + 1])
        return carry

    lax.fori_loop(0, NCHUNK, chunk_body, 0)

    pltpu.sync_copy(pos_v, pos_out.at[wid])
    pltpu.sync_copy(neg_v, neg_out.at[wid])


@jax.jit
def _skipgram(in_table, out_table, in_idx, ctx_idx, neg_idx):
    mesh = plsc.VectorSubcoreMesh(core_axis_name="c", subcore_axis_name="s")
    f = pl.kernel(
        _body,
        out_type=[
            jax.ShapeDtypeStruct((NW, BPW), jnp.float32),
            jax.ShapeDtypeStruct((NW, BPW, NNEG), jnp.float32),
        ],
        mesh=mesh,
        scratch_types=[
            pltpu.VMEM((NCHUNK, CH), jnp.int32),
            pltpu.VMEM((NCHUNK, CH), jnp.int32),
            pltpu.VMEM((NCHUNK, CH, NNEG), jnp.int32),
            pltpu.VMEM((2, NNEG, CH), jnp.int32),
            pltpu.VMEM((2, CH, D), jnp.float32),
            pltpu.VMEM((2, CH, D), jnp.float32),
            pltpu.VMEM((2, NNEG, CH, D), jnp.float32),
            pltpu.VMEM((BPW,), jnp.float32),
            pltpu.VMEM((BPW, NNEG), jnp.float32),
            pltpu.SemaphoreType.DMA((2,)),
        ],
        compiler_params=pltpu.CompilerParams(use_tc_tiling_on_sc=False,
                                             needs_layout_passes=False),
    )
    return f(in_table, out_table, in_idx, ctx_idx, neg_idx)


def kernel(in_table, out_table, inputs, contexts, negatives):
    in_idx = inputs.reshape(NW, NCHUNK, CH)
    ctx_idx = contexts.reshape(NW, NCHUNK, CH)
    neg_idx = negatives.reshape(NW, NCHUNK, CH, NNEG)
    pos, neg = _skipgram(in_table, out_table, in_idx, ctx_idx, neg_idx)
    return pos.reshape(B), neg.reshape(B, NNEG)


# speedup vs baseline: 1.0033x; 1.0033x over previous
import jax
import jax.numpy as jnp
from jax import lax
from jax.experimental import pallas as pl
from jax.experimental.pallas import tpu as pltpu, tpu_sc as plsc

B = 16384
D = 64
NNEG = 20
NC = 2
NS = 16
NW = NC * NS
BPW = B // NW
CH = 32
NCHUNK = BPW // CH
L = 16
GPC = CH // L


def _body(in_table, out_table, in_idx, ctx_idx, neg_idx, pos_out, neg_out,
          in_idx_v, ctx_idx_v, neg_raw_v, neg_idx_t, in_rows, pos_rows,
          neg_rows, pos_v, neg_v, sems):
    wid = lax.axis_index("s") * NC + lax.axis_index("c")

    pltpu.sync_copy(in_idx.at[wid], in_idx_v)
    pltpu.sync_copy(ctx_idx.at[wid], ctx_idx_v)
    pltpu.sync_copy(neg_idx.at[wid], neg_raw_v)

    iota = lax.iota(jnp.int32, L)

    def stage(c, p):
        cvec = jnp.zeros((L,), jnp.int32) + c
        for j in range(NNEG):
            jvec = jnp.full((L,), j, jnp.int32)
            for g in range(GPC):
                rid = iota + (g * L)
                col = plsc.load_gather(neg_raw_v, [cvec, rid, jvec])
                neg_idx_t[p, j, pl.ds(g * L, L)] = col
        cps = [
            pltpu.async_copy(in_table.at[in_idx_v.at[c]], in_rows.at[p],
                             sems.at[p]),
            pltpu.async_copy(out_table.at[ctx_idx_v.at[c]], pos_rows.at[p],
                             sems.at[p]),
        ]
        for j in range(NNEG):
            cps.append(pltpu.async_copy(out_table.at[neg_idx_t.at[p, j]],
                                        neg_rows.at[p, j], sems.at[p]))
        return cps

    def drain(p):
        pltpu.make_async_copy(in_table.at[in_idx_v.at[0]], in_rows.at[p],
                              sems.at[p]).wait()
        pltpu.make_async_copy(out_table.at[ctx_idx_v.at[0]], pos_rows.at[p],
                              sems.at[p]).wait()
        for j in range(NNEG):
            pltpu.make_async_copy(out_table.at[neg_idx_t.at[p, j]],
                                  neg_rows.at[p, j], sems.at[p]).wait()

    stage(0, 0)

    def chunk_body(c, carry):
        p = c & 1
        drain(p)

        @pl.when(c + 1 < NCHUNK)
        def _():
            stage(c + 1, 1 - p)

        pvec = jnp.zeros((L,), jnp.int32) + p

        for g in range(GPC):
            rid = iota + (g * L)

            def d_body(d, accs):
                dvec = (iota + d) & (D - 1)
                inv = plsc.load_gather(in_rows, [pvec, rid, dvec])
                pv = plsc.load_gather(pos_rows, [pvec, rid, dvec])
                new = [accs[0] + inv * pv]
                for j in range(NNEG):
                    jvec = jnp.full((L,), j, jnp.int32)
                    nv = plsc.load_gather(neg_rows, [pvec, jvec, rid, dvec])
                    new.append(accs[j + 1] + inv * nv)
                return tuple(new)

            zeros = tuple(jnp.zeros((L,), jnp.float32)
                          for _ in range(NNEG + 1))
            accs = lax.fori_loop(0, D, d_body, zeros)

            off = c * CH + g * L
            pos_v[pl.ds(off, L)] = accs[0]
            rid_w = iota + off
            for j in range(NNEG):
                jvec = jnp.full((L,), j, jnp.int32)
                plsc.store_scatter(neg_v, [rid_w, jvec], accs[j + 1])
        return carry

    lax.fori_loop(0, NCHUNK, chunk_body, 0)

    pltpu.sync_copy(pos_v, pos_out.at[wid])
    pltpu.sync_copy(neg_v, neg_out.at[wid])


@jax.jit
def _skipgram(in_table, out_table, in_idx, ctx_idx, neg_idx):
    mesh = plsc.VectorSubcoreMesh(core_axis_name="c", subcore_axis_name="s")
    f = pl.kernel(
        _body,
        out_type=[
            jax.ShapeDtypeStruct((NW, BPW), jnp.float32),
            jax.ShapeDtypeStruct((NW, BPW, NNEG), jnp.float32),
        ],
        mesh=mesh,
        scratch_types=[
            pltpu.VMEM((NCHUNK, CH), jnp.int32),
            pltpu.VMEM((NCHUNK, CH), jnp.int32),
            pltpu.VMEM((NCHUNK, CH, NNEG), jnp.int32),
            pltpu.VMEM((2, NNEG, CH), jnp.int32),
            pltpu.VMEM((2, CH, D), jnp.float32),
            pltpu.VMEM((2, CH, D), jnp.float32),
            pltpu.VMEM((2, NNEG, CH, D), jnp.float32),
            pltpu.VMEM((BPW,), jnp.float32),
            pltpu.VMEM((BPW, NNEG), jnp.float32),
            pltpu.SemaphoreType.DMA((2,)),
        ],
        compiler_params=pltpu.CompilerParams(use_tc_tiling_on_sc=False,
                                             needs_layout_passes=False),
    )
    return f(in_table, out_table, in_idx, ctx_idx, neg_idx)


def kernel(in_table, out_table, inputs, contexts, negatives):
    in_idx = inputs.reshape(NW, NCHUNK, CH)
    ctx_idx = contexts.reshape(NW, NCHUNK, CH)
    neg_idx = negatives.reshape(NW, NCHUNK, CH, NNEG)
    pos, neg = _skipgram(in_table, out_table, in_idx, ctx_idx, neg_idx)
    return pos.reshape(B), neg.reshape(B, NNEG)
